# baseline (speedup 1.0000x reference)
import functools

import jax
import jax.numpy as jnp
from jax import lax
from jax.experimental import pallas as pl
from jax.experimental.pallas import tpu as pltpu
from jax.experimental.pallas import tpu_sc as plsc

_NUM_CORES = 2
_NUM_SUBCORES = 16
_NUM_WORKERS = _NUM_CORES * _NUM_SUBCORES
_LANES = 16
_CHUNK = 256


_TCHUNK = 256


def _make_tableprep(n_rows, depth):
    n_main = (n_rows // _TCHUNK) * _TCHUNK
    n_chunks = n_main // _TCHUNK
    tail = n_rows - n_main
    n_iter = -(-n_chunks // _NUM_WORKERS)
    if n_iter % 2:
        n_iter += 1

    mesh = plsc.VectorSubcoreMesh(
        core_axis_name="c",
        subcore_axis_name="s",
        num_cores=_NUM_CORES,
        num_subcores=_NUM_SUBCORES,
    )

    scratch = [
        pltpu.VMEM((depth, _TCHUNK), jnp.float32),
        pltpu.VMEM((depth, _TCHUNK), jnp.float32),
        pltpu.VMEM((_TCHUNK, 2 * depth), jnp.float32),
        pltpu.VMEM((_TCHUNK, 2 * depth), jnp.float32),
        pltpu.SemaphoreType.DMA,
        pltpu.SemaphoreType.DMA,
    ]
    in_types = [jax.ShapeDtypeStruct((depth, n_rows), jnp.float32)]
    if tail:
        in_types.append(jax.ShapeDtypeStruct((tail, 2 * depth), jnp.float32))

    def make_body(has_tail):
        def body_fn(*refs):
            if has_tail:
                (embt_hbm, tail_hbm, tab_hbm,
                 in0, in1, out0, out1, osem0, osem1) = refs
            else:
                (embt_hbm, tab_hbm,
                 in0, in1, out0, out1, osem0, osem1) = refs
            wid = lax.axis_index("s") * _NUM_CORES + lax.axis_index("c")
            inbs = (in0, in1)
            outbs = (out0, out1)
            osems = (osem0, osem1)

            if has_tail:
                @pl.when(wid == _NUM_WORKERS - 1)
                def _():
                    pltpu.sync_copy(tail_hbm,
                                    tab_hbm.at[pl.ds(n_main, tail), :])

            iota = lax.iota(jnp.int32, _LANES)
            diag = [(iota + r) % _LANES for r in range(_LANES)]
            n_kd = depth // _LANES
            n_jr = _TCHUNK // _LANES

            zero = jnp.zeros((_LANES,), jnp.float32)
            for zb in range(2):
                @plsc.parallel_loop(0, _TCHUNK, 1, unroll=8)
                def _(rr):
                    for k in range(depth // _LANES):
                        outbs[zb][rr, pl.ds(depth + k * _LANES, _LANES)] = (
                            zero)

            def transpose(b):
                @plsc.parallel_loop(0, n_jr * n_kd, 1, unroll=4)
                def _(q):
                    jr = q // n_kd
                    kd = q % n_kd
                    rvec = iota + jr * _LANES
                    kvec = jnp.broadcast_to(kd * _LANES, (_LANES,))
                    for r in range(_LANES):
                        dvec = diag[r] + kvec
                        vals = plsc.load_gather(inbs[b], [dvec, rvec])
                        plsc.store_scatter(outbs[b], [rvec, dvec], vals)

            def start_out(i, b):
                r0 = (i * _NUM_WORKERS + wid) * _TCHUNK
                pltpu.async_copy(outbs[b], tab_hbm.at[pl.ds(r0, _TCHUNK), :],
                                 osems[b])

            def wait_out(b):
                pltpu.make_async_copy(outbs[b],
                                      tab_hbm.at[pl.ds(0, _TCHUNK), :],
                                      osems[b]).wait()

            def active(i):
                return i * _NUM_WORKERS + wid < n_chunks

            def body(j, carry):
                for b in range(2):
                    i = 2 * j + b

                    @pl.when((i >= 2) & active(i - 2))
                    def _():
                        wait_out(b)

                    @pl.when(active(i))
                    def _():
                        r0 = (i * _NUM_WORKERS + wid) * _TCHUNK
                        pltpu.sync_copy(embt_hbm.at[:, pl.ds(r0, _TCHUNK)],
                                        inbs[b])
                        transpose(b)
                        start_out(i, b)
                return carry

            lax.fori_loop(0, n_iter // 2, body, 0)

            @pl.when(active(n_iter - 2))
            def _():
                wait_out(0)

            @pl.when(active(n_iter - 1))
            def _():
                wait_out(1)

        return body_fn

    return functools.partial(
        pl.kernel,
        out_type=jax.ShapeDtypeStruct((n_rows, 2 * depth), jnp.float32),
        mesh=mesh,
        scratch_types=scratch,
        compiler_params=pltpu.CompilerParams(
            use_tc_tiling_on_sc=True, needs_layout_passes=False),
    )(make_body(bool(tail)))


def _make_gather(batch, n_fields, depth):
    per_w = batch // _NUM_WORKERS
    n_sub = per_w // _CHUNK
    n_chunks = n_fields * n_sub
    assert per_w % _CHUNK == 0 and n_chunks % 2 == 0

    mesh = plsc.VectorSubcoreMesh(
        core_axis_name="c",
        subcore_axis_name="s",
        num_cores=_NUM_CORES,
        num_subcores=_NUM_SUBCORES,
    )

    @functools.partial(
        pl.kernel,
        out_type=jax.ShapeDtypeStruct((n_fields, depth, batch), jnp.float32),
        mesh=mesh,
        scratch_types=[
            pltpu.VMEM((_CHUNK,), jnp.int32),
            pltpu.VMEM((_CHUNK,), jnp.int32),
            pltpu.VMEM((_CHUNK, 2 * depth), jnp.float32),
            pltpu.VMEM((_CHUNK, 2 * depth), jnp.float32),
            pltpu.VMEM((depth, _CHUNK), jnp.float32),
            pltpu.VMEM((depth, _CHUNK), jnp.float32),
            pltpu.SemaphoreType.DMA,
            pltpu.SemaphoreType.DMA,
            pltpu.SemaphoreType.DMA,
            pltpu.SemaphoreType.DMA,
        ],
        compiler_params=pltpu.CompilerParams(
            use_tc_tiling_on_sc=True, needs_layout_passes=False),
    )
    def gather_kernel(idxt_hbm, table_hbm, out_hbm,
                      idx0, idx1, rows0, rows1, tb0, tb1,
                      gsem0, gsem1, wsem0, wsem1):
        wid = lax.axis_index("s") * _NUM_CORES + lax.axis_index("c")
        b_base = wid * per_w
        idxs = (idx0, idx1)
        rows = (rows0, rows1)
        tbs = (tb0, tb1)
        gsems = (gsem0, gsem1)
        wsems = (wsem0, wsem1)

        def chunk_b0(i):
            return b_base + (i % n_sub) * _CHUNK

        def start_gather(i, b):
            f = i // n_sub
            pltpu.sync_copy(idxt_hbm.at[f, pl.ds(chunk_b0(i), _CHUNK)],
                            idxs[b])
            pltpu.async_copy(table_hbm.at[idxs[b]], rows[b], gsems[b])

        def wait_gather(b):
            pltpu.make_async_copy(table_hbm.at[idxs[b]], rows[b],
                                  gsems[b]).wait()

        def start_write(i, b):
            f = i // n_sub
            pltpu.async_copy(tbs[b],
                             out_hbm.at[f, pl.ds(0, depth),
                                        pl.ds(chunk_b0(i), _CHUNK)],
                             wsems[b])

        def wait_write(b):
            pltpu.make_async_copy(tbs[b],
                                  out_hbm.at[0, pl.ds(0, depth),
                                             pl.ds(b_base, _CHUNK)],
                                  wsems[b]).wait()

        iota = lax.iota(jnp.int32, _LANES)
        diag = [(iota + r) % _LANES for r in range(_LANES)]
        n_kd = depth // _LANES

        def transpose(b):
            @plsc.parallel_loop(0, (_CHUNK // _LANES) * n_kd, 1, unroll=4)
            def _(q):
                jb = q // n_kd
                kd = q % n_kd
                bvec = iota + jb * _LANES
                kvec = jnp.broadcast_to(kd * _LANES, (_LANES,))
                for r in range(_LANES):
                    dvec = diag[r] + kvec
                    vals = plsc.load_gather(rows[b], [bvec, dvec])
                    plsc.store_scatter(tbs[b], [dvec, bvec], vals)

        start_gather(0, 0)

        def body(j, carry):
            for b in range(2):
                i = 2 * j + b
                nb = 1 - b

                @pl.when(i >= 1)
                def _():
                    wait_write(nb)

                @pl.when(i + 1 < n_chunks)
                def _():
                    start_gather(i + 1, nb)

                wait_gather(b)
                transpose(b)
                start_write(i, b)
            return carry

        lax.fori_loop(0, n_chunks // 2, body, 0)
        wait_write(1)

    return gather_kernel


def kernel(indices, embedding):
    batch, n_fields = indices.shape
    n_rows, depth = embedding.shape
    idxt = indices.T.astype(jnp.int32)
    emb_t = embedding.T
    tail = n_rows % _TCHUNK
    prep_args = [emb_t]
    if tail:
        tail128 = jnp.pad(embedding[n_rows - tail:], ((0, 0), (0, depth)))
        prep_args.append(tail128)
    table128 = _make_tableprep(n_rows, depth)(*prep_args)
    out = _make_gather(batch, n_fields, depth)(idxt, table128)
    return jnp.transpose(out, (2, 0, 1))


# speedup vs baseline: 1.6214x; 1.6214x over previous
import functools

import jax
import jax.numpy as jnp
from jax import lax
from jax.experimental import pallas as pl
from jax.experimental.pallas import tpu as pltpu
from jax.experimental.pallas import tpu_sc as plsc

_NUM_CORES = 2
_NUM_SUBCORES = 16
_NUM_WORKERS = _NUM_CORES * _NUM_SUBCORES
_LANES = 16
_CHUNK = 256


_TCHUNK = 256


def _make_tableprep(n_rows, depth):
    n_main = (n_rows // _TCHUNK) * _TCHUNK
    n_chunks = n_main // _TCHUNK
    tail = n_rows - n_main
    n_iter = -(-n_chunks // _NUM_WORKERS)
    if n_iter % 2:
        n_iter += 1

    mesh = plsc.VectorSubcoreMesh(
        core_axis_name="c",
        subcore_axis_name="s",
        num_cores=_NUM_CORES,
        num_subcores=_NUM_SUBCORES,
    )

    scratch = [
        pltpu.VMEM((depth, _TCHUNK), jnp.float32),
        pltpu.VMEM((depth, _TCHUNK), jnp.float32),
        pltpu.VMEM((_TCHUNK, 2 * depth), jnp.float32),
        pltpu.VMEM((_TCHUNK, 2 * depth), jnp.float32),
        pltpu.SemaphoreType.DMA,
        pltpu.SemaphoreType.DMA,
        pltpu.SemaphoreType.DMA,
        pltpu.SemaphoreType.DMA,
    ]
    in_types = [jax.ShapeDtypeStruct((depth, n_rows), jnp.float32)]
    if tail:
        in_types.append(jax.ShapeDtypeStruct((tail, 2 * depth), jnp.float32))

    def make_body(has_tail):
        def body_fn(*refs):
            if has_tail:
                (embt_hbm, tail_hbm, tab_hbm,
                 in0, in1, out0, out1, osem0, osem1, isem0, isem1) = refs
            else:
                (embt_hbm, tab_hbm,
                 in0, in1, out0, out1, osem0, osem1, isem0, isem1) = refs
            wid = lax.axis_index("s") * _NUM_CORES + lax.axis_index("c")
            inbs = (in0, in1)
            outbs = (out0, out1)
            osems = (osem0, osem1)
            isems = (isem0, isem1)

            if has_tail:
                @pl.when(wid == _NUM_WORKERS - 1)
                def _():
                    pltpu.sync_copy(tail_hbm,
                                    tab_hbm.at[pl.ds(n_main, tail), :])

            iota = lax.iota(jnp.int32, _LANES)
            diag = [(iota + r) % _LANES for r in range(_LANES)]
            n_kd = depth // _LANES
            n_jr = _TCHUNK // _LANES


            def transpose(b):
                @plsc.parallel_loop(0, n_jr * n_kd, 1, unroll=4)
                def _(q):
                    jr = q // n_kd
                    kd = q % n_kd
                    rvec = iota + jr * _LANES
                    kvec = jnp.broadcast_to(kd * _LANES, (_LANES,))
                    for r in range(_LANES):
                        dvec = diag[r] + kvec
                        vals = plsc.load_gather(inbs[b], [dvec, rvec])
                        plsc.store_scatter(outbs[b], [rvec, dvec], vals)

            def start_out(i, b):
                r0 = (i * _NUM_WORKERS + wid) * _TCHUNK
                pltpu.async_copy(outbs[b], tab_hbm.at[pl.ds(r0, _TCHUNK), :],
                                 osems[b])

            def start_in(i, b):
                r0 = (i * _NUM_WORKERS + wid) * _TCHUNK
                pltpu.async_copy(embt_hbm.at[:, pl.ds(r0, _TCHUNK)],
                                 inbs[b], isems[b])

            def wait_in(b):
                pltpu.make_async_copy(embt_hbm.at[:, pl.ds(0, _TCHUNK)],
                                      inbs[b], isems[b]).wait()

            def wait_out(b):
                pltpu.make_async_copy(outbs[b],
                                      tab_hbm.at[pl.ds(0, _TCHUNK), :],
                                      osems[b]).wait()

            def active(i):
                return i * _NUM_WORKERS + wid < n_chunks

            @pl.when(active(0))
            def _():
                start_in(0, 0)

            def body(j, carry):
                for b in range(2):
                    i = 2 * j + b
                    nb = 1 - b

                    @pl.when(active(i + 1))
                    def _():
                        start_in(i + 1, nb)

                    @pl.when((i >= 2) & active(i - 2))
                    def _():
                        wait_out(b)

                    @pl.when(active(i))
                    def _():
                        wait_in(b)
                        transpose(b)
                        start_out(i, b)
                return carry

            lax.fori_loop(0, n_iter // 2, body, 0)

            @pl.when(active(n_iter - 2))
            def _():
                wait_out(0)

            @pl.when(active(n_iter - 1))
            def _():
                wait_out(1)

        return body_fn

    return functools.partial(
        pl.kernel,
        out_type=jax.ShapeDtypeStruct((n_rows, 2 * depth), jnp.float32),
        mesh=mesh,
        scratch_types=scratch,
        compiler_params=pltpu.CompilerParams(
            use_tc_tiling_on_sc=True, needs_layout_passes=False),
    )(make_body(bool(tail)))


def _make_gather(batch, n_fields, depth):
    per_w = batch // _NUM_WORKERS
    n_sub = per_w // _CHUNK
    n_chunks = n_fields * n_sub
    assert per_w % _CHUNK == 0 and n_chunks % 2 == 0

    mesh = plsc.VectorSubcoreMesh(
        core_axis_name="c",
        subcore_axis_name="s",
        num_cores=_NUM_CORES,
        num_subcores=_NUM_SUBCORES,
    )

    @functools.partial(
        pl.kernel,
        out_type=jax.ShapeDtypeStruct((n_fields, depth, batch), jnp.float32),
        mesh=mesh,
        scratch_types=[
            pltpu.VMEM((_CHUNK,), jnp.int32),
            pltpu.VMEM((_CHUNK,), jnp.int32),
            pltpu.VMEM((_CHUNK, 2 * depth), jnp.float32),
            pltpu.VMEM((_CHUNK, 2 * depth), jnp.float32),
            pltpu.VMEM((depth, _CHUNK), jnp.float32),
            pltpu.VMEM((depth, _CHUNK), jnp.float32),
            pltpu.SemaphoreType.DMA,
            pltpu.SemaphoreType.DMA,
            pltpu.SemaphoreType.DMA,
            pltpu.SemaphoreType.DMA,
        ],
        compiler_params=pltpu.CompilerParams(
            use_tc_tiling_on_sc=True, needs_layout_passes=False),
    )
    def gather_kernel(idxt_hbm, table_hbm, out_hbm,
                      idx0, idx1, rows0, rows1, tb0, tb1,
                      gsem0, gsem1, wsem0, wsem1):
        wid = lax.axis_index("s") * _NUM_CORES + lax.axis_index("c")
        b_base = wid * per_w
        idxs = (idx0, idx1)
        rows = (rows0, rows1)
        tbs = (tb0, tb1)
        gsems = (gsem0, gsem1)
        wsems = (wsem0, wsem1)

        def chunk_b0(i):
            return b_base + (i % n_sub) * _CHUNK

        def start_gather(i, b):
            f = i // n_sub
            pltpu.sync_copy(idxt_hbm.at[f, pl.ds(chunk_b0(i), _CHUNK)],
                            idxs[b])
            pltpu.async_copy(table_hbm.at[idxs[b]], rows[b], gsems[b])

        def wait_gather(b):
            pltpu.make_async_copy(table_hbm.at[idxs[b]], rows[b],
                                  gsems[b]).wait()

        def start_write(i, b):
            f = i // n_sub
            pltpu.async_copy(tbs[b],
                             out_hbm.at[f, pl.ds(0, depth),
                                        pl.ds(chunk_b0(i), _CHUNK)],
                             wsems[b])

        def wait_write(b):
            pltpu.make_async_copy(tbs[b],
                                  out_hbm.at[0, pl.ds(0, depth),
                                             pl.ds(b_base, _CHUNK)],
                                  wsems[b]).wait()

        iota = lax.iota(jnp.int32, _LANES)
        diag = [(iota + r) % _LANES for r in range(_LANES)]
        n_kd = depth // _LANES

        def transpose(b):
            @plsc.parallel_loop(0, (_CHUNK // _LANES) * n_kd, 1, unroll=4)
            def _(q):
                jb = q // n_kd
                kd = q % n_kd
                bvec = iota + jb * _LANES
                kvec = jnp.broadcast_to(kd * _LANES, (_LANES,))
                for r in range(_LANES):
                    dvec = diag[r] + kvec
                    vals = plsc.load_gather(rows[b], [bvec, dvec])
                    plsc.store_scatter(tbs[b], [dvec, bvec], vals)

        start_gather(0, 0)

        def body(j, carry):
            for b in range(2):
                i = 2 * j + b
                nb = 1 - b

                @pl.when(i >= 1)
                def _():
                    wait_write(nb)

                @pl.when(i + 1 < n_chunks)
                def _():
                    start_gather(i + 1, nb)

                wait_gather(b)
                transpose(b)
                start_write(i, b)
            return carry

        lax.fori_loop(0, n_chunks // 2, body, 0)
        wait_write(1)

    return gather_kernel


def kernel(indices, embedding):
    batch, n_fields = indices.shape
    n_rows, depth = embedding.shape
    idxt = indices.T.astype(jnp.int32)
    emb_t = embedding.T
    tail = n_rows % _TCHUNK
    prep_args = [emb_t]
    if tail:
        tail128 = jnp.pad(embedding[n_rows - tail:], ((0, 0), (0, depth)))
        prep_args.append(tail128)
    table128 = _make_tableprep(n_rows, depth)(*prep_args)
    out = _make_gather(batch, n_fields, depth)(idxt, table128)
    return jnp.transpose(out, (2, 0, 1))


# speedup vs baseline: 1.6597x; 1.0236x over previous
import functools

import jax
import jax.numpy as jnp
from jax import lax
from jax.experimental import pallas as pl
from jax.experimental.pallas import tpu as pltpu
from jax.experimental.pallas import tpu_sc as plsc

_NUM_CORES = 2
_NUM_SUBCORES = 16
_NUM_WORKERS = _NUM_CORES * _NUM_SUBCORES
_LANES = 16
_CHUNK = 256


_TCHUNK = 256


def _make_tableprep(n_rows, depth):
    n_main = (n_rows // _TCHUNK) * _TCHUNK
    n_chunks = n_main // _TCHUNK
    tail = n_rows - n_main
    n_iter = -(-n_chunks // _NUM_WORKERS)
    if n_iter % 2:
        n_iter += 1

    mesh = plsc.VectorSubcoreMesh(
        core_axis_name="c",
        subcore_axis_name="s",
        num_cores=_NUM_CORES,
        num_subcores=_NUM_SUBCORES,
    )

    scratch = [
        pltpu.VMEM((depth, _TCHUNK), jnp.float32),
        pltpu.VMEM((depth, _TCHUNK), jnp.float32),
        pltpu.VMEM((_TCHUNK, 2 * depth), jnp.float32),
        pltpu.VMEM((_TCHUNK, 2 * depth), jnp.float32),
        pltpu.SemaphoreType.DMA,
        pltpu.SemaphoreType.DMA,
        pltpu.SemaphoreType.DMA,
        pltpu.SemaphoreType.DMA,
    ]
    in_types = [jax.ShapeDtypeStruct((depth, n_rows), jnp.float32)]
    if tail:
        in_types.append(jax.ShapeDtypeStruct((tail, 2 * depth), jnp.float32))

    def make_body(has_tail):
        def body_fn(*refs):
            if has_tail:
                (embt_hbm, tail_hbm, tab_hbm,
                 in0, in1, out0, out1, osem0, osem1, isem0, isem1) = refs
            else:
                (embt_hbm, tab_hbm,
                 in0, in1, out0, out1, osem0, osem1, isem0, isem1) = refs
            wid = lax.axis_index("s") * _NUM_CORES + lax.axis_index("c")
            inbs = (in0, in1)
            outbs = (out0, out1)
            osems = (osem0, osem1)
            isems = (isem0, isem1)

            if has_tail:
                @pl.when(wid == _NUM_WORKERS - 1)
                def _():
                    pltpu.sync_copy(tail_hbm,
                                    tab_hbm.at[pl.ds(n_main, tail), :])

            iota = lax.iota(jnp.int32, _LANES)
            diag = [(iota + r) % _LANES for r in range(_LANES)]
            n_kd = depth // _LANES
            n_jr = _TCHUNK // _LANES


            def transpose(b):
                @plsc.parallel_loop(0, n_jr * n_kd, 1, unroll=8)
                def _(q):
                    jr = q // n_kd
                    kd = q % n_kd
                    rvec = iota + jr * _LANES
                    kvec = jnp.broadcast_to(kd * _LANES, (_LANES,))
                    for r in range(_LANES):
                        dvec = diag[r] + kvec
                        vals = plsc.load_gather(inbs[b], [dvec, rvec])
                        plsc.store_scatter(outbs[b], [rvec, dvec], vals)

            def start_out(i, b):
                r0 = (i * _NUM_WORKERS + wid) * _TCHUNK
                pltpu.async_copy(outbs[b], tab_hbm.at[pl.ds(r0, _TCHUNK), :],
                                 osems[b])

            def start_in(i, b):
                r0 = (i * _NUM_WORKERS + wid) * _TCHUNK
                pltpu.async_copy(embt_hbm.at[:, pl.ds(r0, _TCHUNK)],
                                 inbs[b], isems[b])

            def wait_in(b):
                pltpu.make_async_copy(embt_hbm.at[:, pl.ds(0, _TCHUNK)],
                                      inbs[b], isems[b]).wait()

            def wait_out(b):
                pltpu.make_async_copy(outbs[b],
                                      tab_hbm.at[pl.ds(0, _TCHUNK), :],
                                      osems[b]).wait()

            def active(i):
                return i * _NUM_WORKERS + wid < n_chunks

            @pl.when(active(0))
            def _():
                start_in(0, 0)

            def body(j, carry):
                for b in range(2):
                    i = 2 * j + b
                    nb = 1 - b

                    @pl.when(active(i + 1))
                    def _():
                        start_in(i + 1, nb)

                    @pl.when((i >= 2) & active(i - 2))
                    def _():
                        wait_out(b)

                    @pl.when(active(i))
                    def _():
                        wait_in(b)
                        transpose(b)
                        start_out(i, b)
                return carry

            lax.fori_loop(0, n_iter // 2, body, 0)

            @pl.when(active(n_iter - 2))
            def _():
                wait_out(0)

            @pl.when(active(n_iter - 1))
            def _():
                wait_out(1)

        return body_fn

    return functools.partial(
        pl.kernel,
        out_type=jax.ShapeDtypeStruct((n_rows, 2 * depth), jnp.float32),
        mesh=mesh,
        scratch_types=scratch,
        compiler_params=pltpu.CompilerParams(
            use_tc_tiling_on_sc=True, needs_layout_passes=False),
    )(make_body(bool(tail)))


def _make_gather(batch, n_fields, depth):
    per_w = batch // _NUM_WORKERS
    n_sub = per_w // _CHUNK
    n_chunks = n_fields * n_sub
    assert per_w % _CHUNK == 0 and n_chunks % 2 == 0

    mesh = plsc.VectorSubcoreMesh(
        core_axis_name="c",
        subcore_axis_name="s",
        num_cores=_NUM_CORES,
        num_subcores=_NUM_SUBCORES,
    )

    @functools.partial(
        pl.kernel,
        out_type=jax.ShapeDtypeStruct((n_fields, depth, batch), jnp.float32),
        mesh=mesh,
        scratch_types=[
            pltpu.VMEM((_CHUNK,), jnp.int32),
            pltpu.VMEM((_CHUNK,), jnp.int32),
            pltpu.VMEM((_CHUNK, 2 * depth), jnp.float32),
            pltpu.VMEM((_CHUNK, 2 * depth), jnp.float32),
            pltpu.VMEM((depth, _CHUNK), jnp.float32),
            pltpu.VMEM((depth, _CHUNK), jnp.float32),
            pltpu.SemaphoreType.DMA,
            pltpu.SemaphoreType.DMA,
            pltpu.SemaphoreType.DMA,
            pltpu.SemaphoreType.DMA,
        ],
        compiler_params=pltpu.CompilerParams(
            use_tc_tiling_on_sc=True, needs_layout_passes=False),
    )
    def gather_kernel(idxt_hbm, table_hbm, out_hbm,
                      idx0, idx1, rows0, rows1, tb0, tb1,
                      gsem0, gsem1, wsem0, wsem1):
        wid = lax.axis_index("s") * _NUM_CORES + lax.axis_index("c")
        b_base = wid * per_w
        idxs = (idx0, idx1)
        rows = (rows0, rows1)
        tbs = (tb0, tb1)
        gsems = (gsem0, gsem1)
        wsems = (wsem0, wsem1)

        def chunk_b0(i):
            return b_base + (i % n_sub) * _CHUNK

        def start_gather(i, b):
            f = i // n_sub
            pltpu.sync_copy(idxt_hbm.at[f, pl.ds(chunk_b0(i), _CHUNK)],
                            idxs[b])
            pltpu.async_copy(table_hbm.at[idxs[b]], rows[b], gsems[b])

        def wait_gather(b):
            pltpu.make_async_copy(table_hbm.at[idxs[b]], rows[b],
                                  gsems[b]).wait()

        def start_write(i, b):
            f = i // n_sub
            pltpu.async_copy(tbs[b],
                             out_hbm.at[f, pl.ds(0, depth),
                                        pl.ds(chunk_b0(i), _CHUNK)],
                             wsems[b])

        def wait_write(b):
            pltpu.make_async_copy(tbs[b],
                                  out_hbm.at[0, pl.ds(0, depth),
                                             pl.ds(b_base, _CHUNK)],
                                  wsems[b]).wait()

        iota = lax.iota(jnp.int32, _LANES)
        diag = [(iota + r) % _LANES for r in range(_LANES)]
        n_kd = depth // _LANES

        def transpose(b):
            @plsc.parallel_loop(0, (_CHUNK // _LANES) * n_kd, 1, unroll=4)
            def _(q):
                jb = q // n_kd
                kd = q % n_kd
                bvec = iota + jb * _LANES
                kvec = jnp.broadcast_to(kd * _LANES, (_LANES,))
                for r in range(_LANES):
                    dvec = diag[r] + kvec
                    vals = plsc.load_gather(rows[b], [bvec, dvec])
                    plsc.store_scatter(tbs[b], [dvec, bvec], vals)

        start_gather(0, 0)

        def body(j, carry):
            for b in range(2):
                i = 2 * j + b
                nb = 1 - b

                @pl.when(i >= 1)
                def _():
                    wait_write(nb)

                @pl.when(i + 1 < n_chunks)
                def _():
                    start_gather(i + 1, nb)

                wait_gather(b)
                transpose(b)
                start_write(i, b)
            return carry

        lax.fori_loop(0, n_chunks // 2, body, 0)
        wait_write(1)

    return gather_kernel


def kernel(indices, embedding):
    batch, n_fields = indices.shape
    n_rows, depth = embedding.shape
    idxt = indices.T.astype(jnp.int32)
    emb_t = embedding.T
    tail = n_rows % _TCHUNK
    prep_args = [emb_t]
    if tail:
        tail128 = jnp.pad(embedding[n_rows - tail:], ((0, 0), (0, depth)))
        prep_args.append(tail128)
    table128 = _make_tableprep(n_rows, depth)(*prep_args)
    out = _make_gather(batch, n_fields, depth)(idxt, table128)
    return jnp.transpose(out, (2, 0, 1))


# speedup vs baseline: 1.9394x; 1.1685x over previous
import functools

import jax
import jax.numpy as jnp
from jax import lax
from jax.experimental import pallas as pl
from jax.experimental.pallas import tpu as pltpu
from jax.experimental.pallas import tpu_sc as plsc

_NUM_CORES = 2
_NUM_SUBCORES = 16
_NUM_WORKERS = _NUM_CORES * _NUM_SUBCORES
_LANES = 16
_CHUNK = 256


_TCHUNK = 256


def _make_tableprep(n_rows, depth):
    n_main = (n_rows // _TCHUNK) * _TCHUNK
    n_chunks = n_main // _TCHUNK
    tail = n_rows - n_main
    n_iter = -(-n_chunks // _NUM_WORKERS)
    if n_iter % 2:
        n_iter += 1

    mesh = plsc.VectorSubcoreMesh(
        core_axis_name="c",
        subcore_axis_name="s",
        num_cores=_NUM_CORES,
        num_subcores=_NUM_SUBCORES,
    )

    scratch = [
        pltpu.VMEM((depth, _TCHUNK), jnp.float32),
        pltpu.VMEM((depth, _TCHUNK), jnp.float32),
        pltpu.VMEM((_TCHUNK, 2 * depth), jnp.float32),
        pltpu.VMEM((_TCHUNK, 2 * depth), jnp.float32),
        pltpu.SemaphoreType.DMA,
        pltpu.SemaphoreType.DMA,
        pltpu.SemaphoreType.DMA,
        pltpu.SemaphoreType.DMA,
    ]
    in_types = [jax.ShapeDtypeStruct((depth, n_rows), jnp.float32)]
    if tail:
        in_types.append(jax.ShapeDtypeStruct((tail, 2 * depth), jnp.float32))

    def make_body(has_tail):
        def body_fn(*refs):
            if has_tail:
                (embt_hbm, tail_hbm, tab_hbm,
                 in0, in1, out0, out1, osem0, osem1, isem0, isem1) = refs
            else:
                (embt_hbm, tab_hbm,
                 in0, in1, out0, out1, osem0, osem1, isem0, isem1) = refs
            wid = lax.axis_index("s") * _NUM_CORES + lax.axis_index("c")
            inbs = (in0, in1)
            outbs = (out0, out1)
            osems = (osem0, osem1)
            isems = (isem0, isem1)

            if has_tail:
                @pl.when(wid == _NUM_WORKERS - 1)
                def _():
                    pltpu.sync_copy(tail_hbm,
                                    tab_hbm.at[pl.ds(n_main, tail), :])

            iota = lax.iota(jnp.int32, _LANES)
            diag = [(iota + r) % _LANES for r in range(_LANES)]
            n_kd = depth // _LANES
            n_jr = _TCHUNK // _LANES


            def transpose(b):
                @plsc.parallel_loop(0, n_jr * n_kd, 1, unroll=8)
                def _(q):
                    jr = q // n_kd
                    kd = q % n_kd
                    rvec = iota + jr * _LANES
                    kvec = jnp.broadcast_to(kd * _LANES, (_LANES,))
                    for r in range(_LANES):
                        dvec = diag[r] + kvec
                        vals = plsc.load_gather(inbs[b], [dvec, rvec])
                        plsc.store_scatter(outbs[b], [rvec, dvec], vals)

            def start_out(i, b):
                r0 = (i * _NUM_WORKERS + wid) * _TCHUNK
                pltpu.async_copy(outbs[b], tab_hbm.at[pl.ds(r0, _TCHUNK), :],
                                 osems[b])

            def start_in(i, b):
                r0 = (i * _NUM_WORKERS + wid) * _TCHUNK
                pltpu.async_copy(embt_hbm.at[:, pl.ds(r0, _TCHUNK)],
                                 inbs[b], isems[b])

            def wait_in(b):
                pltpu.make_async_copy(embt_hbm.at[:, pl.ds(0, _TCHUNK)],
                                      inbs[b], isems[b]).wait()

            def wait_out(b):
                pltpu.make_async_copy(outbs[b],
                                      tab_hbm.at[pl.ds(0, _TCHUNK), :],
                                      osems[b]).wait()

            def active(i):
                return i * _NUM_WORKERS + wid < n_chunks

            @pl.when(active(0))
            def _():
                start_in(0, 0)

            def body(j, carry):
                for b in range(2):
                    i = 2 * j + b
                    nb = 1 - b

                    @pl.when(active(i + 1))
                    def _():
                        start_in(i + 1, nb)

                    @pl.when((i >= 2) & active(i - 2))
                    def _():
                        wait_out(b)

                    @pl.when(active(i))
                    def _():
                        wait_in(b)
                        transpose(b)
                        start_out(i, b)
                return carry

            lax.fori_loop(0, n_iter // 2, body, 0)

            @pl.when(active(n_iter - 2))
            def _():
                wait_out(0)

            @pl.when(active(n_iter - 1))
            def _():
                wait_out(1)

        return body_fn

    return functools.partial(
        pl.kernel,
        out_type=jax.ShapeDtypeStruct((n_rows, 2 * depth), jnp.float32),
        mesh=mesh,
        scratch_types=scratch,
        compiler_params=pltpu.CompilerParams(
            use_tc_tiling_on_sc=True, needs_layout_passes=False),
    )(make_body(bool(tail)))


def _make_gather(batch, n_fields, depth):
    per_w = batch // _NUM_WORKERS
    n_sub = per_w // _CHUNK
    n_chunks = n_fields * n_sub
    assert per_w % _CHUNK == 0 and n_chunks % 2 == 0

    mesh = plsc.VectorSubcoreMesh(
        core_axis_name="c",
        subcore_axis_name="s",
        num_cores=_NUM_CORES,
        num_subcores=_NUM_SUBCORES,
    )

    @functools.partial(
        pl.kernel,
        out_type=jax.ShapeDtypeStruct((n_fields, depth, batch), jnp.float32),
        mesh=mesh,
        scratch_types=[
            pltpu.VMEM((_CHUNK,), jnp.int32),
            pltpu.VMEM((_CHUNK,), jnp.int32),
            pltpu.VMEM((_CHUNK, 2 * depth), jnp.float32),
            pltpu.VMEM((_CHUNK, 2 * depth), jnp.float32),
            pltpu.VMEM((depth, _CHUNK), jnp.float32),
            pltpu.VMEM((depth, _CHUNK), jnp.float32),
            pltpu.SemaphoreType.DMA,
            pltpu.SemaphoreType.DMA,
            pltpu.SemaphoreType.DMA,
            pltpu.SemaphoreType.DMA,
        ],
        compiler_params=pltpu.CompilerParams(
            use_tc_tiling_on_sc=True, needs_layout_passes=False),
    )
    def gather_kernel(idxt_hbm, table_hbm, out_hbm,
                      idx0, idx1, rows0, rows1, tb0, tb1,
                      gsem0, gsem1, wsem0, wsem1):
        wid = lax.axis_index("s") * _NUM_CORES + lax.axis_index("c")
        b_base = wid * per_w
        idxs = (idx0, idx1)
        rows = (rows0, rows1)
        tbs = (tb0, tb1)
        gsems = (gsem0, gsem1)
        wsems = (wsem0, wsem1)

        def chunk_b0(i):
            return b_base + (i % n_sub) * _CHUNK

        def start_gather(i, b):
            f = i // n_sub
            pltpu.sync_copy(idxt_hbm.at[f, pl.ds(chunk_b0(i), _CHUNK)],
                            idxs[b])
            pltpu.async_copy(table_hbm.at[idxs[b]], rows[b], gsems[b])

        def wait_gather(b):
            pltpu.make_async_copy(table_hbm.at[idxs[b]], rows[b],
                                  gsems[b]).wait()

        def start_write(i, b):
            f = i // n_sub
            pltpu.async_copy(tbs[b],
                             out_hbm.at[f, pl.ds(0, depth),
                                        pl.ds(chunk_b0(i), _CHUNK)],
                             wsems[b])

        def wait_write(b):
            pltpu.make_async_copy(tbs[b],
                                  out_hbm.at[0, pl.ds(0, depth),
                                             pl.ds(b_base, _CHUNK)],
                                  wsems[b]).wait()

        iota = lax.iota(jnp.int32, _LANES)
        diag = [(iota + r) % _LANES for r in range(_LANES)]
        n_kd = depth // _LANES

        def transpose(b):
            @plsc.parallel_loop(0, (_CHUNK // _LANES) * n_kd, 1, unroll=8)
            def _(q):
                jb = q // n_kd
                kd = q % n_kd
                bvec = iota + jb * _LANES
                kvec = jnp.broadcast_to(kd * _LANES, (_LANES,))
                for r in range(_LANES):
                    dvec = diag[r] + kvec
                    vals = plsc.load_gather(rows[b], [bvec, dvec])
                    plsc.store_scatter(tbs[b], [dvec, bvec], vals)

        start_gather(0, 0)

        def body(j, carry):
            for b in range(2):
                i = 2 * j + b
                nb = 1 - b

                @pl.when(i >= 1)
                def _():
                    wait_write(nb)

                @pl.when(i + 1 < n_chunks)
                def _():
                    start_gather(i + 1, nb)

                wait_gather(b)
                transpose(b)
                start_write(i, b)
            return carry

        lax.fori_loop(0, n_chunks // 2, body, 0)
        wait_write(1)

    return gather_kernel


def kernel(indices, embedding):
    batch, n_fields = indices.shape
    n_rows, depth = embedding.shape
    idxt = indices.T.astype(jnp.int32)
    emb_t = embedding.T
    tail = n_rows % _TCHUNK
    prep_args = [emb_t]
    if tail:
        tail128 = jnp.pad(embedding[n_rows - tail:], ((0, 0), (0, depth)))
        prep_args.append(tail128)
    table128 = _make_tableprep(n_rows, depth)(*prep_args)
    out = _make_gather(batch, n_fields, depth)(idxt, table128)
    return jnp.transpose(out, (2, 0, 1))
